# baseline (device time: 224564 ns/iter reference)
import jax
import jax.numpy as jnp
from jax import lax
from jax.experimental import pallas as pl
from jax.experimental.pallas import tpu as pltpu

N_DEV = 8


def kernel(x, w_mat, scale_x, scale_w):
    m_per, k = x.shape
    _, n_per = w_mat.shape

    x8 = x.astype(jnp.float8_e4m3fn)
    w8 = w_mat.astype(jnp.float8_e4m3fn)

    def body(x_ref, w_ref, sx_ref, sw_ref, out_ref, comm_ref, send_sems, recv_sems):
        my = lax.axis_index("i")
        left = lax.rem(my + N_DEV - 1, N_DEV)
        right = lax.rem(my + 1, N_DEV)

        barrier = pltpu.get_barrier_semaphore()
        pl.semaphore_signal(barrier, inc=1, device_id=(left,),
                            device_id_type=pl.DeviceIdType.MESH)
        pl.semaphore_signal(barrier, inc=1, device_id=(right,),
                            device_id_type=pl.DeviceIdType.MESH)
        pl.semaphore_wait(barrier, 2)

        scale = sx_ref[0] * sw_ref[0]

        comm_ref[0] = x_ref[...]
        out_ref[pl.ds(my * m_per, m_per), :] = (
            jnp.dot(x_ref[...], w_ref[...], preferred_element_type=jnp.float32)
            * scale
        )

        for h in range(N_DEV - 1):
            send_slot = h % 2
            recv_slot = (h + 1) % 2
            rdma = pltpu.make_async_remote_copy(
                src_ref=comm_ref.at[send_slot],
                dst_ref=comm_ref.at[recv_slot],
                send_sem=send_sems.at[send_slot],
                recv_sem=recv_sems.at[recv_slot],
                device_id=(right,),
                device_id_type=pl.DeviceIdType.MESH,
            )
            rdma.start()
            rdma.wait()
            origin = lax.rem(my - h - 1 + N_DEV, N_DEV)
            out_ref[pl.ds(origin * m_per, m_per), :] = (
                jnp.dot(comm_ref[recv_slot], w_ref[...],
                        preferred_element_type=jnp.float32)
                * scale
            )

    return pl.pallas_call(
        body,
        out_shape=jax.ShapeDtypeStruct((N_DEV * m_per, n_per), jnp.float32),
        in_specs=[
            pl.BlockSpec(memory_space=pltpu.VMEM),
            pl.BlockSpec(memory_space=pltpu.VMEM),
            pl.BlockSpec(memory_space=pltpu.SMEM),
            pl.BlockSpec(memory_space=pltpu.SMEM),
        ],
        out_specs=pl.BlockSpec(memory_space=pltpu.VMEM),
        scratch_shapes=[
            pltpu.VMEM((2, m_per, k), jnp.float8_e4m3fn),
            pltpu.SemaphoreType.DMA((2,)),
            pltpu.SemaphoreType.DMA((2,)),
        ],
        compiler_params=pltpu.CompilerParams(collective_id=0),
    )(x8, w8, scale_x, scale_w)


# device time: 130907 ns/iter; 1.7154x vs baseline; 1.7154x over previous
import jax
import jax.numpy as jnp
from jax import lax
from jax.experimental import pallas as pl
from jax.experimental.pallas import tpu as pltpu

N_DEV = 8


def kernel(x, w_mat, scale_x, scale_w):
    m_per, k = x.shape
    _, n_per = w_mat.shape
    m_half = m_per // 2

    x8 = x.astype(jnp.float8_e4m3fn)
    w8 = w_mat.astype(jnp.float8_e4m3fn)

    def body(x_ref, w_ref, sx_ref, sw_ref, out_ref,
             cw_ref, ccw_ref, cw_send, cw_recv, ccw_send, ccw_recv):
        my = lax.axis_index("i")
        left = lax.rem(my + N_DEV - 1, N_DEV)
        right = lax.rem(my + 1, N_DEV)

        barrier = pltpu.get_barrier_semaphore()
        pl.semaphore_signal(barrier, inc=1, device_id=(left,),
                            device_id_type=pl.DeviceIdType.MESH)
        pl.semaphore_signal(barrier, inc=1, device_id=(right,),
                            device_id_type=pl.DeviceIdType.MESH)
        pl.semaphore_wait(barrier, 2)

        scale = sx_ref[0] * sw_ref[0]

        cw_ref[0] = x_ref[:m_half, :]
        ccw_ref[0] = x_ref[m_half:, :]

        def make_cw(h):
            return pltpu.make_async_remote_copy(
                src_ref=cw_ref.at[h],
                dst_ref=cw_ref.at[h + 1],
                send_sem=cw_send.at[h],
                recv_sem=cw_recv.at[h],
                device_id=(right,),
                device_id_type=pl.DeviceIdType.MESH,
            )

        def make_ccw(h):
            return pltpu.make_async_remote_copy(
                src_ref=ccw_ref.at[h],
                dst_ref=ccw_ref.at[h + 1],
                send_sem=ccw_send.at[h],
                recv_sem=ccw_recv.at[h],
                device_id=(left,),
                device_id_type=pl.DeviceIdType.MESH,
            )

        cw_rdmas = [make_cw(h) for h in range(N_DEV - 1)]
        ccw_rdmas = [make_ccw(h) for h in range(N_DEV - 1)]

        cw_rdmas[0].start()
        ccw_rdmas[0].start()
        out_ref[pl.ds(my * m_per, m_per), :] = (
            jnp.dot(x_ref[...], w_ref[...], preferred_element_type=jnp.float32)
            * scale
        )

        for h in range(N_DEV - 1):
            cw_rdmas[h].wait_recv()
            ccw_rdmas[h].wait_recv()
            if h + 1 < N_DEV - 1:
                cw_rdmas[h + 1].start()
                ccw_rdmas[h + 1].start()
            o_top = lax.rem(my - h - 1 + N_DEV, N_DEV)
            o_bot = lax.rem(my + h + 1, N_DEV)
            out_ref[pl.ds(o_top * m_per, m_half), :] = (
                jnp.dot(cw_ref[h + 1], w_ref[...],
                        preferred_element_type=jnp.float32) * scale
            )
            out_ref[pl.ds(o_bot * m_per + m_half, m_half), :] = (
                jnp.dot(ccw_ref[h + 1], w_ref[...],
                        preferred_element_type=jnp.float32) * scale
            )

        for h in range(N_DEV - 1):
            cw_rdmas[h].wait_send()
            ccw_rdmas[h].wait_send()

    return pl.pallas_call(
        body,
        out_shape=jax.ShapeDtypeStruct((N_DEV * m_per, n_per), jnp.float32),
        in_specs=[
            pl.BlockSpec(memory_space=pltpu.VMEM),
            pl.BlockSpec(memory_space=pltpu.VMEM),
            pl.BlockSpec(memory_space=pltpu.SMEM),
            pl.BlockSpec(memory_space=pltpu.SMEM),
        ],
        out_specs=pl.BlockSpec(memory_space=pltpu.VMEM),
        scratch_shapes=[
            pltpu.VMEM((N_DEV, m_half, k), jnp.float8_e4m3fn),
            pltpu.VMEM((N_DEV, m_half, k), jnp.float8_e4m3fn),
            pltpu.SemaphoreType.DMA((N_DEV - 1,)),
            pltpu.SemaphoreType.DMA((N_DEV - 1,)),
            pltpu.SemaphoreType.DMA((N_DEV - 1,)),
            pltpu.SemaphoreType.DMA((N_DEV - 1,)),
        ],
        compiler_params=pltpu.CompilerParams(collective_id=0),
    )(x8, w8, scale_x, scale_w)


# device time: 127582 ns/iter; 1.7602x vs baseline; 1.0261x over previous
import jax
import jax.numpy as jnp
from jax import lax
from jax.experimental import pallas as pl
from jax.experimental.pallas import tpu as pltpu

N_DEV = 8


def kernel(x, w_mat, scale_x, scale_w):
    m_per, k = x.shape
    _, n_per = w_mat.shape
    m_half = m_per // 2

    def body(x_ref, w_ref, sx_ref, sw_ref, out_ref,
             w8_ref, cw_ref, ccw_ref, cw_send, cw_recv, ccw_send, ccw_recv):
        my = lax.axis_index("i")
        left = lax.rem(my + N_DEV - 1, N_DEV)
        right = lax.rem(my + 1, N_DEV)

        barrier = pltpu.get_barrier_semaphore()
        pl.semaphore_signal(barrier, inc=1, device_id=(left,),
                            device_id_type=pl.DeviceIdType.MESH)
        pl.semaphore_signal(barrier, inc=1, device_id=(right,),
                            device_id_type=pl.DeviceIdType.MESH)
        pl.semaphore_wait(barrier, 2)

        scale = sx_ref[0] * sw_ref[0]

        cw_ref[0] = x_ref[:m_half, :].astype(jnp.float8_e4m3fn)
        ccw_ref[0] = x_ref[m_half:, :].astype(jnp.float8_e4m3fn)

        def make_cw(h):
            return pltpu.make_async_remote_copy(
                src_ref=cw_ref.at[h],
                dst_ref=cw_ref.at[h + 1],
                send_sem=cw_send.at[h],
                recv_sem=cw_recv.at[h],
                device_id=(right,),
                device_id_type=pl.DeviceIdType.MESH,
            )

        def make_ccw(h):
            return pltpu.make_async_remote_copy(
                src_ref=ccw_ref.at[h],
                dst_ref=ccw_ref.at[h + 1],
                send_sem=ccw_send.at[h],
                recv_sem=ccw_recv.at[h],
                device_id=(left,),
                device_id_type=pl.DeviceIdType.MESH,
            )

        cw_rdmas = [make_cw(h) for h in range(N_DEV - 1)]
        ccw_rdmas = [make_ccw(h) for h in range(N_DEV - 1)]

        cw_rdmas[0].start()
        ccw_rdmas[0].start()
        w8_ref[...] = w_ref[...].astype(jnp.float8_e4m3fn)
        out_ref[pl.ds(my * m_per, m_half), :] = (
            jnp.dot(cw_ref[0], w8_ref[...], preferred_element_type=jnp.float32)
            * scale
        )
        out_ref[pl.ds(my * m_per + m_half, m_half), :] = (
            jnp.dot(ccw_ref[0], w8_ref[...], preferred_element_type=jnp.float32)
            * scale
        )

        for h in range(N_DEV - 1):
            cw_rdmas[h].wait_recv()
            ccw_rdmas[h].wait_recv()
            if h + 1 < N_DEV - 1:
                cw_rdmas[h + 1].start()
                ccw_rdmas[h + 1].start()
            o_top = lax.rem(my - h - 1 + N_DEV, N_DEV)
            o_bot = lax.rem(my + h + 1, N_DEV)
            out_ref[pl.ds(o_top * m_per, m_half), :] = (
                jnp.dot(cw_ref[h + 1], w8_ref[...],
                        preferred_element_type=jnp.float32) * scale
            )
            out_ref[pl.ds(o_bot * m_per + m_half, m_half), :] = (
                jnp.dot(ccw_ref[h + 1], w8_ref[...],
                        preferred_element_type=jnp.float32) * scale
            )

        for h in range(N_DEV - 1):
            cw_rdmas[h].wait_send()
            ccw_rdmas[h].wait_send()

    return pl.pallas_call(
        body,
        out_shape=jax.ShapeDtypeStruct((N_DEV * m_per, n_per), jnp.float32),
        in_specs=[
            pl.BlockSpec(memory_space=pltpu.VMEM),
            pl.BlockSpec(memory_space=pltpu.VMEM),
            pl.BlockSpec(memory_space=pltpu.SMEM),
            pl.BlockSpec(memory_space=pltpu.SMEM),
        ],
        out_specs=pl.BlockSpec(memory_space=pltpu.VMEM),
        scratch_shapes=[
            pltpu.VMEM((k, n_per), jnp.float8_e4m3fn),
            pltpu.VMEM((N_DEV, m_half, k), jnp.float8_e4m3fn),
            pltpu.VMEM((N_DEV, m_half, k), jnp.float8_e4m3fn),
            pltpu.SemaphoreType.DMA((N_DEV - 1,)),
            pltpu.SemaphoreType.DMA((N_DEV - 1,)),
            pltpu.SemaphoreType.DMA((N_DEV - 1,)),
            pltpu.SemaphoreType.DMA((N_DEV - 1,)),
        ],
        compiler_params=pltpu.CompilerParams(
            collective_id=0,
            vmem_limit_bytes=100 * 1024 * 1024,
        ),
    )(x, w_mat, scale_x, scale_w)


# device time: 124279 ns/iter; 1.8069x vs baseline; 1.0266x over previous
import jax
import jax.numpy as jnp
from jax import lax
from jax.experimental import pallas as pl
from jax.experimental.pallas import tpu as pltpu

N_DEV = 8


def kernel(x, w_mat, scale_x, scale_w):
    m_per, k = x.shape
    _, n_per = w_mat.shape
    m_half = m_per // 2

    def body(x_ref, w_ref, sx_ref, sw_ref, out_ref,
             w8_ref, cw_ref, ccw_ref, cw_send, cw_recv, ccw_send, ccw_recv):
        my = lax.axis_index("i")

        def ring(p):
            return jnp.where(p < 4, p, 11 - p)

        pos = ring(my)
        right = ring(lax.rem(pos + 1, N_DEV))
        left = ring(lax.rem(pos + N_DEV - 1, N_DEV))

        barrier = pltpu.get_barrier_semaphore()
        pl.semaphore_signal(barrier, inc=1, device_id=(left,),
                            device_id_type=pl.DeviceIdType.MESH)
        pl.semaphore_signal(barrier, inc=1, device_id=(right,),
                            device_id_type=pl.DeviceIdType.MESH)
        pl.semaphore_wait(barrier, 2)

        scale = sx_ref[0] * sw_ref[0]

        cw_ref[0] = x_ref[:m_half, :].astype(jnp.float8_e4m3fn)
        ccw_ref[0] = x_ref[m_half:, :].astype(jnp.float8_e4m3fn)

        def make_cw(h):
            return pltpu.make_async_remote_copy(
                src_ref=cw_ref.at[h],
                dst_ref=cw_ref.at[h + 1],
                send_sem=cw_send.at[h],
                recv_sem=cw_recv.at[h],
                device_id=(right,),
                device_id_type=pl.DeviceIdType.MESH,
            )

        def make_ccw(h):
            return pltpu.make_async_remote_copy(
                src_ref=ccw_ref.at[h],
                dst_ref=ccw_ref.at[h + 1],
                send_sem=ccw_send.at[h],
                recv_sem=ccw_recv.at[h],
                device_id=(left,),
                device_id_type=pl.DeviceIdType.MESH,
            )

        cw_rdmas = [make_cw(h) for h in range(N_DEV - 1)]
        ccw_rdmas = [make_ccw(h) for h in range(N_DEV - 1)]

        cw_rdmas[0].start()
        ccw_rdmas[0].start()
        w8_ref[...] = w_ref[...].astype(jnp.float8_e4m3fn)
        out_ref[pl.ds(my * m_per, m_half), :] = (
            jnp.dot(cw_ref[0], w8_ref[...], preferred_element_type=jnp.float32)
            * scale
        )
        out_ref[pl.ds(my * m_per + m_half, m_half), :] = (
            jnp.dot(ccw_ref[0], w8_ref[...], preferred_element_type=jnp.float32)
            * scale
        )

        for h in range(N_DEV - 1):
            cw_rdmas[h].wait_recv()
            ccw_rdmas[h].wait_recv()
            if h + 1 < N_DEV - 1:
                cw_rdmas[h + 1].start()
                ccw_rdmas[h + 1].start()
            o_top = ring(lax.rem(pos - h - 1 + N_DEV, N_DEV))
            o_bot = ring(lax.rem(pos + h + 1, N_DEV))
            out_ref[pl.ds(o_top * m_per, m_half), :] = (
                jnp.dot(cw_ref[h + 1], w8_ref[...],
                        preferred_element_type=jnp.float32) * scale
            )
            out_ref[pl.ds(o_bot * m_per + m_half, m_half), :] = (
                jnp.dot(ccw_ref[h + 1], w8_ref[...],
                        preferred_element_type=jnp.float32) * scale
            )

        for h in range(N_DEV - 1):
            cw_rdmas[h].wait_send()
            ccw_rdmas[h].wait_send()

    return pl.pallas_call(
        body,
        out_shape=jax.ShapeDtypeStruct((N_DEV * m_per, n_per), jnp.float32),
        in_specs=[
            pl.BlockSpec(memory_space=pltpu.VMEM),
            pl.BlockSpec(memory_space=pltpu.VMEM),
            pl.BlockSpec(memory_space=pltpu.SMEM),
            pl.BlockSpec(memory_space=pltpu.SMEM),
        ],
        out_specs=pl.BlockSpec(memory_space=pltpu.VMEM),
        scratch_shapes=[
            pltpu.VMEM((k, n_per), jnp.float8_e4m3fn),
            pltpu.VMEM((N_DEV, m_half, k), jnp.float8_e4m3fn),
            pltpu.VMEM((N_DEV, m_half, k), jnp.float8_e4m3fn),
            pltpu.SemaphoreType.DMA((N_DEV - 1,)),
            pltpu.SemaphoreType.DMA((N_DEV - 1,)),
            pltpu.SemaphoreType.DMA((N_DEV - 1,)),
            pltpu.SemaphoreType.DMA((N_DEV - 1,)),
        ],
        compiler_params=pltpu.CompilerParams(
            collective_id=0,
            vmem_limit_bytes=100 * 1024 * 1024,
        ),
    )(x, w_mat, scale_x, scale_w)


# device time: 90549 ns/iter; 2.4800x vs baseline; 1.3725x over previous
import jax
import jax.numpy as jnp
from jax import lax
from jax.experimental import pallas as pl
from jax.experimental.pallas import tpu as pltpu

N_DEV = 8
PART_ROWS = (192, 160, 160)
PART_OFF = (0, 192, 352)
DIM_MASKS = (1, 3, 4)
PART_DIMS = tuple(tuple(DIM_MASKS[(p + r) % 3] for r in range(3))
                  for p in range(3))


def _origin_mask(p, j):
    m = 0
    for r in range(3):
        if j & (1 << r):
            m ^= PART_DIMS[p][r]
    return m


def kernel(x, w_mat, scale_x, scale_w):
    m_per, k = x.shape
    _, n_per = w_mat.shape

    def body(x_ref, w_ref, sx_ref, sw_ref, out_ref,
             w8_ref, b0, b1, b2, s0, r0, s1, r1, s2, r2):
        bufs = (b0, b1, b2)
        ssems = (s0, s1, s2)
        rsems = (r0, r1, r2)

        my = lax.axis_index("i")

        barrier = pltpu.get_barrier_semaphore()
        for m in DIM_MASKS:
            pl.semaphore_signal(barrier, inc=1,
                                device_id=(jnp.bitwise_xor(my, m),),
                                device_id_type=pl.DeviceIdType.MESH)
        pl.semaphore_wait(barrier, 3)

        scale = sx_ref[0] * sw_ref[0]

        def rdma(p, src_slot, dst_slot, dim_mask):
            return pltpu.make_async_remote_copy(
                src_ref=bufs[p].at[src_slot],
                dst_ref=bufs[p].at[dst_slot],
                send_sem=ssems[p].at[dst_slot - 1],
                recv_sem=rsems[p].at[dst_slot - 1],
                device_id=(jnp.bitwise_xor(my, dim_mask),),
                device_id_type=pl.DeviceIdType.MESH,
            )

        desc = {}
        for p in range(3):
            d0m, d1m, d2m = PART_DIMS[p]
            desc[(p, 1)] = rdma(p, 0, 1, d0m)
            desc[(p, 2)] = rdma(p, 0, 2, d1m)
            desc[(p, 3)] = rdma(p, 1, 3, d1m)
            for j in range(4):
                desc[(p, 4 + j)] = rdma(p, j, 4 + j, d2m)

        def gemm(p, j):
            origin = jnp.bitwise_xor(my, _origin_mask(p, j))
            out_ref[pl.ds(origin * m_per + PART_OFF[p], PART_ROWS[p]), :] = (
                jnp.dot(bufs[p][j], w8_ref[...],
                        preferred_element_type=jnp.float32) * scale
            )

        for p in range(3):
            lo, rows = PART_OFF[p], PART_ROWS[p]
            bufs[p][0] = x_ref[lo:lo + rows, :].astype(jnp.float8_e4m3fn)
            desc[(p, 1)].start()
            desc[(p, 2)].start()
            desc[(p, 4)].start()

        w8_ref[...] = w_ref[...].astype(jnp.float8_e4m3fn)
        for p in range(3):
            gemm(p, 0)

        for p in range(3):
            desc[(p, 1)].wait_recv()
            desc[(p, 3)].start()
            desc[(p, 5)].start()
            gemm(p, 1)

        for p in range(3):
            desc[(p, 2)].wait_recv()
            desc[(p, 6)].start()
            gemm(p, 2)
        for p in range(3):
            desc[(p, 3)].wait_recv()
            desc[(p, 7)].start()
            gemm(p, 3)

        for j in range(4, 8):
            for p in range(3):
                desc[(p, j)].wait_recv()
                gemm(p, j)

        for p in range(3):
            for j in range(1, 8):
                desc[(p, j)].wait_send()

    return pl.pallas_call(
        body,
        out_shape=jax.ShapeDtypeStruct((N_DEV * m_per, n_per), jnp.float32),
        in_specs=[
            pl.BlockSpec(memory_space=pltpu.VMEM),
            pl.BlockSpec(memory_space=pltpu.VMEM),
            pl.BlockSpec(memory_space=pltpu.SMEM),
            pl.BlockSpec(memory_space=pltpu.SMEM),
        ],
        out_specs=pl.BlockSpec(memory_space=pltpu.VMEM),
        scratch_shapes=[
            pltpu.VMEM((k, n_per), jnp.float8_e4m3fn),
            pltpu.VMEM((N_DEV, PART_ROWS[0], k), jnp.float8_e4m3fn),
            pltpu.VMEM((N_DEV, PART_ROWS[1], k), jnp.float8_e4m3fn),
            pltpu.VMEM((N_DEV, PART_ROWS[2], k), jnp.float8_e4m3fn),
            pltpu.SemaphoreType.DMA((7,)), pltpu.SemaphoreType.DMA((7,)),
            pltpu.SemaphoreType.DMA((7,)), pltpu.SemaphoreType.DMA((7,)),
            pltpu.SemaphoreType.DMA((7,)), pltpu.SemaphoreType.DMA((7,)),
        ],
        compiler_params=pltpu.CompilerParams(
            collective_id=0,
            vmem_limit_bytes=100 * 1024 * 1024,
        ),
    )(x, w_mat, scale_x, scale_w)
